# Initial kernel scaffold; baseline (speedup 1.0000x reference)
import jax
import jax.numpy as jnp
from jax.experimental import pallas as pl


def kernel(x, principal_dir, curvature, density, normals, linearity, Wq, Wk, Wv, Wo, geo_w, geo_b):
    raise NotImplementedError("write your pallas kernel here")


# speedup vs baseline: 7.0796x; 7.0796x over previous
import functools
import numpy as np
import jax
import jax.numpy as jnp
from jax import lax
from jax.experimental import pallas as pl
from jax.experimental.pallas import tpu as pltpu

_HEADS = 8
_KNN = 16
_ROWS = 256
_SUB = 8


def _proj_kernel(x_ref, pd_ref, nrm_ref, cur_ref, den_ref, lin_ref,
                 wq_ref, wk_ref, wv_ref, gw_ref, gb_ref,
                 q_ref, k_ref, v_ref, rowop_ref, colop_ref, gbias_ref):
    xb = x_ref[0]
    q_ref[0] = jnp.dot(xb, wq_ref[...], preferred_element_type=jnp.float32)
    k_ref[0] = jnp.dot(xb, wk_ref[...], preferred_element_type=jnp.float32)
    v_ref[0] = jnp.dot(xb, wv_ref[...], preferred_element_type=jnp.float32)
    geo = jnp.concatenate([pd_ref[0], nrm_ref[0], cur_ref[0],
                           den_ref[0], lin_ref[0]], axis=1)
    r = geo.shape[0]
    sq = jnp.sum(geo * geo, axis=1, keepdims=True)
    zeros7 = jnp.zeros((r, 7), jnp.float32)
    zeros6 = jnp.zeros((r, 6), jnp.float32)
    rowop_ref[0] = jnp.concatenate([-2.0 * geo, zeros7], axis=1)
    colop_ref[0] = jnp.concatenate([geo, sq, zeros6], axis=1)
    geo16 = jnp.concatenate([geo, zeros7], axis=1)
    gbias_ref[0] = (jnp.dot(geo16, gw_ref[...],
                            preferred_element_type=jnp.float32)
                    + gb_ref[...])


def _attn_kernel(q_ref, k_ref, v_ref, rowop_ref, colop_ref, gbias_ref,
                 wo_ref, o_ref, dscr, mscr):
    hi = jax.lax.Precision.HIGHEST
    rb = q_ref.shape[1]
    nn = k_ref.shape[1]
    h = _HEADS
    dh = q_ref.shape[2] // h
    g = lax.dot_general(rowop_ref[0], colop_ref[0], (((1,), (1,)), ((), ())),
                        preferred_element_type=jnp.float32)
    e9 = (lax.broadcasted_iota(jnp.int32, (1, 16), 1) == 9
          ).astype(jnp.float32)
    sqrow = lax.dot_general(e9, colop_ref[0], (((1,), (1,)), ((), ())),
                            precision=hi, preferred_element_type=jnp.float32)
    dscr[...] = g + sqrow

    inf = jnp.float32(np.inf)

    def sub(j, carry):
        dw = dscr[pl.ds(j * _SUB, _SUB), :]
        m = jnp.min(dw, axis=1, keepdims=True)
        for _ in range(_KNN - 1):
            dw = jnp.where(dw <= m, inf, dw)
            m = jnp.min(dw, axis=1, keepdims=True)
        orig = dscr[pl.ds(j * _SUB, _SUB), :]
        mscr[pl.ds(j * _SUB, _SUB), :] = jnp.where(orig <= m, 0.0, -1e30)
        return carry

    lax.fori_loop(0, rb // _SUB, sub, 0, unroll=False)

    eye = jnp.eye(h, dtype=jnp.float32)
    biast = lax.dot_general(eye, gbias_ref[0], (((1,), (1,)), ((), ())),
                            precision=hi, preferred_element_type=jnp.float32)
    q = q_ref[0]
    k = k_ref[0]
    v = v_ref[0]
    scale = jnp.float32(1.0 / np.sqrt(dh))
    outs = []
    for hh in range(h):
        qh = q[:, hh * dh:(hh + 1) * dh]
        kh = k[:, hh * dh:(hh + 1) * dh]
        vh = v[:, hh * dh:(hh + 1) * dh]
        s = lax.dot_general(qh, kh, (((1,), (1,)), ((), ())),
                            preferred_element_type=jnp.float32) * scale
        s = s + biast[hh:hh + 1, :] + mscr[...]
        mx = jnp.max(s, axis=1, keepdims=True)
        p = jnp.exp(s - mx)
        denom = jnp.sum(p, axis=1, keepdims=True)
        p = p / denom
        outs.append(jnp.dot(p, vh, preferred_element_type=jnp.float32))
    ob = jnp.concatenate(outs, axis=1)
    o_ref[0] = jnp.dot(ob, wo_ref[...], preferred_element_type=jnp.float32)


@jax.jit
def kernel(x, principal_dir, curvature, density, normals, linearity,
           Wq, Wk, Wv, Wo, geo_w, geo_b):
    b, n, d = x.shape
    r = _ROWS
    nb = n // r
    f32 = jnp.float32
    gw16 = jnp.zeros((16, _HEADS), f32).at[:geo_w.shape[0]].set(geo_w)
    gb2 = geo_b.reshape(1, _HEADS)

    grid = (b, nb)
    row3 = lambda bi, i: (bi, i, 0)
    full3 = lambda bi, i: (bi, 0, 0)
    wmap = lambda bi, i: (0, 0)

    q, k, v, rowop, colop, gbias = pl.pallas_call(
        _proj_kernel,
        grid=grid,
        in_specs=[
            pl.BlockSpec((1, r, d), row3),
            pl.BlockSpec((1, r, 3), row3),
            pl.BlockSpec((1, r, 3), row3),
            pl.BlockSpec((1, r, 1), row3),
            pl.BlockSpec((1, r, 1), row3),
            pl.BlockSpec((1, r, 1), row3),
            pl.BlockSpec((d, d), wmap),
            pl.BlockSpec((d, d), wmap),
            pl.BlockSpec((d, d), wmap),
            pl.BlockSpec((16, _HEADS), wmap),
            pl.BlockSpec((1, _HEADS), wmap),
        ],
        out_specs=[
            pl.BlockSpec((1, r, d), row3),
            pl.BlockSpec((1, r, d), row3),
            pl.BlockSpec((1, r, d), row3),
            pl.BlockSpec((1, r, 16), row3),
            pl.BlockSpec((1, r, 16), row3),
            pl.BlockSpec((1, r, _HEADS), row3),
        ],
        out_shape=[
            jax.ShapeDtypeStruct((b, n, d), f32),
            jax.ShapeDtypeStruct((b, n, d), f32),
            jax.ShapeDtypeStruct((b, n, d), f32),
            jax.ShapeDtypeStruct((b, n, 16), f32),
            jax.ShapeDtypeStruct((b, n, 16), f32),
            jax.ShapeDtypeStruct((b, n, _HEADS), f32),
        ],
        compiler_params=pltpu.CompilerParams(
            dimension_semantics=("parallel", "parallel")),
    )(x, principal_dir, normals, curvature, density, linearity,
      Wq, Wk, Wv, gw16, gb2)

    out = pl.pallas_call(
        _attn_kernel,
        grid=grid,
        in_specs=[
            pl.BlockSpec((1, r, d), row3),
            pl.BlockSpec((1, n, d), full3),
            pl.BlockSpec((1, n, d), full3),
            pl.BlockSpec((1, r, 16), row3),
            pl.BlockSpec((1, n, 16), full3),
            pl.BlockSpec((1, n, _HEADS), full3),
            pl.BlockSpec((d, d), wmap),
        ],
        out_specs=pl.BlockSpec((1, r, d), row3),
        out_shape=jax.ShapeDtypeStruct((b, n, d), f32),
        scratch_shapes=[
            pltpu.VMEM((r, n), f32),
            pltpu.VMEM((r, n), f32),
        ],
        compiler_params=pltpu.CompilerParams(
            dimension_semantics=("parallel", "arbitrary")),
    )(q, k, v, rowop, colop, gbias, Wo)
    return out


# speedup vs baseline: 7.9771x; 1.1268x over previous
import functools
import numpy as np
import jax
import jax.numpy as jnp
from jax import lax
from jax.experimental import pallas as pl
from jax.experimental.pallas import tpu as pltpu

_HEADS = 8
_KNN = 16
_ROWS = 256
_SUB = 8


def _proj_kernel(x_ref, pd_ref, nrm_ref, cur_ref, den_ref, lin_ref,
                 wq_ref, wk_ref, wv_ref, gw_ref, gb_ref,
                 q_ref, k_ref, v_ref, rowop_ref, colop_ref, gbias_ref):
    xb = x_ref[0]
    q_ref[0] = jnp.dot(xb, wq_ref[...], preferred_element_type=jnp.float32)
    k_ref[0] = jnp.dot(xb, wk_ref[...], preferred_element_type=jnp.float32)
    v_ref[0] = jnp.dot(xb, wv_ref[...], preferred_element_type=jnp.float32)
    geo = jnp.concatenate([pd_ref[0], nrm_ref[0], cur_ref[0],
                           den_ref[0], lin_ref[0]], axis=1)
    r = geo.shape[0]
    sq = jnp.sum(geo * geo, axis=1, keepdims=True)
    zeros7 = jnp.zeros((r, 7), jnp.float32)
    zeros6 = jnp.zeros((r, 6), jnp.float32)
    rowop_ref[0] = jnp.concatenate([-2.0 * geo, zeros7], axis=1)
    colop_ref[0] = jnp.concatenate([geo, sq, zeros6], axis=1)
    geo16 = jnp.concatenate([geo, zeros7], axis=1)
    gbias_ref[0] = (jnp.dot(geo16, gw_ref[...],
                            preferred_element_type=jnp.float32)
                    + gb_ref[...]) * jnp.float32(
                        np.sqrt(x_ref.shape[2] / _HEADS))


def _attn_kernel(q_ref, k_ref, v_ref, rowop_ref, colop_ref, gbias_ref,
                 wo_ref, o_ref, dscr, mscr):
    hi = jax.lax.Precision.HIGHEST
    rb = q_ref.shape[1]
    nn = k_ref.shape[1]
    h = _HEADS
    dh = q_ref.shape[2] // h
    g = lax.dot_general(rowop_ref[0], colop_ref[0], (((1,), (1,)), ((), ())),
                        preferred_element_type=jnp.float32)
    e9 = (lax.broadcasted_iota(jnp.int32, (1, 16), 1) == 9
          ).astype(jnp.float32)
    sqrow = lax.dot_general(e9, colop_ref[0], (((1,), (1,)), ((), ())),
                            precision=hi, preferred_element_type=jnp.float32)
    dscr[...] = g + sqrow

    inf = jnp.float32(np.inf)

    def sub(j, carry):
        dw = dscr[pl.ds(j * _SUB, _SUB), :]
        m = jnp.min(dw, axis=1, keepdims=True)
        for _ in range(_KNN - 1):
            dw = jnp.where(dw <= m, inf, dw)
            m = jnp.min(dw, axis=1, keepdims=True)
        orig = dscr[pl.ds(j * _SUB, _SUB), :]
        mscr[pl.ds(j * _SUB, _SUB), :] = jnp.where(orig <= m, 0.0, -1e30)
        return carry

    lax.fori_loop(0, rb // _SUB, sub, 0, unroll=False)

    q = q_ref[0]
    k = k_ref[0]
    v = v_ref[0]
    gbias = gbias_ref[0]
    scale = jnp.float32(1.0 / np.sqrt(dh))
    ones_r = jnp.ones((rb, 1), jnp.float32)
    ones_n = jnp.ones((nn, 1), jnp.float32)
    msk = mscr[...]
    outs = []
    for hh in range(h):
        qa = jnp.concatenate([q[:, hh * dh:(hh + 1) * dh], ones_r], axis=1)
        ka = jnp.concatenate([k[:, hh * dh:(hh + 1) * dh],
                              gbias[:, hh:hh + 1]], axis=1)
        va = jnp.concatenate([v[:, hh * dh:(hh + 1) * dh], ones_n], axis=1)
        t = lax.dot_general(qa, ka, (((1,), (1,)), ((), ())),
                            preferred_element_type=jnp.float32)
        p = jnp.exp((t + msk) * scale)
        oa = lax.dot_general(p, va, (((1,), (0,)), ((), ())),
                             preferred_element_type=jnp.float32)
        outs.append(oa[:, :dh] / oa[:, dh:dh + 1])
    ob = jnp.concatenate(outs, axis=1)
    o_ref[0] = jnp.dot(ob, wo_ref[...], preferred_element_type=jnp.float32)


@jax.jit
def kernel(x, principal_dir, curvature, density, normals, linearity,
           Wq, Wk, Wv, Wo, geo_w, geo_b):
    b, n, d = x.shape
    r = _ROWS
    nb = n // r
    f32 = jnp.float32
    gw16 = jnp.zeros((16, _HEADS), f32).at[:geo_w.shape[0]].set(geo_w)
    gb2 = geo_b.reshape(1, _HEADS)

    grid = (b, nb)
    row3 = lambda bi, i: (bi, i, 0)
    full3 = lambda bi, i: (bi, 0, 0)
    wmap = lambda bi, i: (0, 0)

    q, k, v, rowop, colop, gbias = pl.pallas_call(
        _proj_kernel,
        grid=grid,
        in_specs=[
            pl.BlockSpec((1, r, d), row3),
            pl.BlockSpec((1, r, 3), row3),
            pl.BlockSpec((1, r, 3), row3),
            pl.BlockSpec((1, r, 1), row3),
            pl.BlockSpec((1, r, 1), row3),
            pl.BlockSpec((1, r, 1), row3),
            pl.BlockSpec((d, d), wmap),
            pl.BlockSpec((d, d), wmap),
            pl.BlockSpec((d, d), wmap),
            pl.BlockSpec((16, _HEADS), wmap),
            pl.BlockSpec((1, _HEADS), wmap),
        ],
        out_specs=[
            pl.BlockSpec((1, r, d), row3),
            pl.BlockSpec((1, r, d), row3),
            pl.BlockSpec((1, r, d), row3),
            pl.BlockSpec((1, r, 16), row3),
            pl.BlockSpec((1, r, 16), row3),
            pl.BlockSpec((1, r, _HEADS), row3),
        ],
        out_shape=[
            jax.ShapeDtypeStruct((b, n, d), f32),
            jax.ShapeDtypeStruct((b, n, d), f32),
            jax.ShapeDtypeStruct((b, n, d), f32),
            jax.ShapeDtypeStruct((b, n, 16), f32),
            jax.ShapeDtypeStruct((b, n, 16), f32),
            jax.ShapeDtypeStruct((b, n, _HEADS), f32),
        ],
        compiler_params=pltpu.CompilerParams(
            dimension_semantics=("parallel", "parallel")),
    )(x, principal_dir, normals, curvature, density, linearity,
      Wq, Wk, Wv, gw16, gb2)

    out = pl.pallas_call(
        _attn_kernel,
        grid=grid,
        in_specs=[
            pl.BlockSpec((1, r, d), row3),
            pl.BlockSpec((1, n, d), full3),
            pl.BlockSpec((1, n, d), full3),
            pl.BlockSpec((1, r, 16), row3),
            pl.BlockSpec((1, n, 16), full3),
            pl.BlockSpec((1, n, _HEADS), full3),
            pl.BlockSpec((d, d), wmap),
        ],
        out_specs=pl.BlockSpec((1, r, d), row3),
        out_shape=jax.ShapeDtypeStruct((b, n, d), f32),
        scratch_shapes=[
            pltpu.VMEM((r, n), f32),
            pltpu.VMEM((r, n), f32),
        ],
        compiler_params=pltpu.CompilerParams(
            dimension_semantics=("parallel", "arbitrary")),
    )(q, k, v, rowop, colop, gbias, Wo)
    return out


# speedup vs baseline: 17.8501x; 2.2377x over previous
import functools
import numpy as np
import jax
import jax.numpy as jnp
from jax import lax
from jax.experimental import pallas as pl
from jax.experimental.pallas import tpu as pltpu

_HEADS = 8
_KNN = 16
_ROWS = 256
_SUB = 8


def _proj_kernel(x_ref, pd_ref, nrm_ref, cur_ref, den_ref, lin_ref,
                 wq_ref, wk_ref, wv_ref, gw_ref, gb_ref,
                 q_ref, k_ref, v_ref, rowop_ref, colop_ref, gbias_ref):
    xb = x_ref[0]
    q_ref[0] = jnp.dot(xb, wq_ref[...], preferred_element_type=jnp.float32)
    k_ref[0] = jnp.dot(xb, wk_ref[...], preferred_element_type=jnp.float32)
    v_ref[0] = jnp.dot(xb, wv_ref[...], preferred_element_type=jnp.float32)
    geo = jnp.concatenate([pd_ref[0], nrm_ref[0], cur_ref[0],
                           den_ref[0], lin_ref[0]], axis=1)
    r = geo.shape[0]
    sq = jnp.sum(geo * geo, axis=1, keepdims=True)
    zeros7 = jnp.zeros((r, 7), jnp.float32)
    zeros6 = jnp.zeros((r, 6), jnp.float32)
    rowop_ref[0] = jnp.concatenate([-2.0 * geo, zeros7], axis=1)
    colop_ref[0] = jnp.concatenate([geo, sq, zeros6], axis=1)
    geo16 = jnp.concatenate([geo, zeros7], axis=1)
    gbias_ref[0] = (jnp.dot(geo16, gw_ref[...],
                            preferred_element_type=jnp.float32)
                    + gb_ref[...]) * jnp.float32(
                        np.sqrt(x_ref.shape[2] / _HEADS))


def _attn_kernel(q_ref, k_ref, v_ref, rowop_ref, colop_ref, gbias_ref,
                 wo_ref, o_ref, dscr, mscr):
    hi = jax.lax.Precision.HIGHEST
    rb = q_ref.shape[1]
    nn = k_ref.shape[1]
    h = _HEADS
    dh = q_ref.shape[2] // h
    g = lax.dot_general(rowop_ref[0], colop_ref[0], (((1,), (1,)), ((), ())),
                        preferred_element_type=jnp.float32)
    e9 = (lax.broadcasted_iota(jnp.int32, (1, 16), 1) == 9
          ).astype(jnp.float32)
    sqrow = lax.dot_general(e9, colop_ref[0], (((1,), (1,)), ((), ())),
                            precision=hi, preferred_element_type=jnp.float32)
    dscr[...] = g + sqrow

    inf = jnp.float32(np.inf)

    def sub(j, carry):
        dw = dscr[pl.ds(j * _SUB, _SUB), :]
        m = jnp.min(dw, axis=1, keepdims=True)
        for _ in range(_KNN - 1):
            dw = jnp.where(dw <= m, inf, dw)
            m = jnp.min(dw, axis=1, keepdims=True)
        orig = dscr[pl.ds(j * _SUB, _SUB), :]
        mscr[pl.ds(j * _SUB, _SUB), :] = jnp.where(orig <= m, 0.0, -1e30)
        return carry

    lax.fori_loop(0, rb // _SUB, sub, 0, unroll=4)

    q = q_ref[0]
    k = k_ref[0]
    v = v_ref[0]
    gbias = gbias_ref[0]
    scale = jnp.float32(1.0 / np.sqrt(dh))
    ones_r = jnp.ones((rb, 1), jnp.float32)
    ones_n = jnp.ones((nn, 1), jnp.float32)
    msk = mscr[...]
    outs = []
    for hh in range(h):
        qa = jnp.concatenate([q[:, hh * dh:(hh + 1) * dh], ones_r], axis=1)
        ka = jnp.concatenate([k[:, hh * dh:(hh + 1) * dh],
                              gbias[:, hh:hh + 1]], axis=1)
        va = jnp.concatenate([v[:, hh * dh:(hh + 1) * dh], ones_n], axis=1)
        t = lax.dot_general(qa, ka, (((1,), (1,)), ((), ())),
                            preferred_element_type=jnp.float32)
        p = jnp.exp((t + msk) * scale)
        oa = lax.dot_general(p, va, (((1,), (0,)), ((), ())),
                             preferred_element_type=jnp.float32)
        outs.append(oa[:, :dh] / oa[:, dh:dh + 1])
    ob = jnp.concatenate(outs, axis=1)
    o_ref[0] = jnp.dot(ob, wo_ref[...], preferred_element_type=jnp.float32)


@jax.jit
def kernel(x, principal_dir, curvature, density, normals, linearity,
           Wq, Wk, Wv, Wo, geo_w, geo_b):
    b, n, d = x.shape
    r = _ROWS
    nb = n // r
    f32 = jnp.float32
    gw16 = jnp.zeros((16, _HEADS), f32).at[:geo_w.shape[0]].set(geo_w)
    gb2 = geo_b.reshape(1, _HEADS)

    grid = (b, nb)
    row3 = lambda bi, i: (bi, i, 0)
    full3 = lambda bi, i: (bi, 0, 0)
    wmap = lambda bi, i: (0, 0)

    q, k, v, rowop, colop, gbias = pl.pallas_call(
        _proj_kernel,
        grid=grid,
        in_specs=[
            pl.BlockSpec((1, r, d), row3),
            pl.BlockSpec((1, r, 3), row3),
            pl.BlockSpec((1, r, 3), row3),
            pl.BlockSpec((1, r, 1), row3),
            pl.BlockSpec((1, r, 1), row3),
            pl.BlockSpec((1, r, 1), row3),
            pl.BlockSpec((d, d), wmap),
            pl.BlockSpec((d, d), wmap),
            pl.BlockSpec((d, d), wmap),
            pl.BlockSpec((16, _HEADS), wmap),
            pl.BlockSpec((1, _HEADS), wmap),
        ],
        out_specs=[
            pl.BlockSpec((1, r, d), row3),
            pl.BlockSpec((1, r, d), row3),
            pl.BlockSpec((1, r, d), row3),
            pl.BlockSpec((1, r, 16), row3),
            pl.BlockSpec((1, r, 16), row3),
            pl.BlockSpec((1, r, _HEADS), row3),
        ],
        out_shape=[
            jax.ShapeDtypeStruct((b, n, d), f32),
            jax.ShapeDtypeStruct((b, n, d), f32),
            jax.ShapeDtypeStruct((b, n, d), f32),
            jax.ShapeDtypeStruct((b, n, 16), f32),
            jax.ShapeDtypeStruct((b, n, 16), f32),
            jax.ShapeDtypeStruct((b, n, _HEADS), f32),
        ],
        compiler_params=pltpu.CompilerParams(
            dimension_semantics=("parallel", "parallel")),
    )(x, principal_dir, normals, curvature, density, linearity,
      Wq, Wk, Wv, gw16, gb2)

    out = pl.pallas_call(
        _attn_kernel,
        grid=grid,
        in_specs=[
            pl.BlockSpec((1, r, d), row3),
            pl.BlockSpec((1, n, d), full3),
            pl.BlockSpec((1, n, d), full3),
            pl.BlockSpec((1, r, 16), row3),
            pl.BlockSpec((1, n, 16), full3),
            pl.BlockSpec((1, n, _HEADS), full3),
            pl.BlockSpec((d, d), wmap),
        ],
        out_specs=pl.BlockSpec((1, r, d), row3),
        out_shape=jax.ShapeDtypeStruct((b, n, d), f32),
        scratch_shapes=[
            pltpu.VMEM((r, n), f32),
            pltpu.VMEM((r, n), f32),
        ],
        compiler_params=pltpu.CompilerParams(
            dimension_semantics=("parallel", "arbitrary")),
    )(q, k, v, rowop, colop, gbias, Wo)
    return out


# speedup vs baseline: 22.0541x; 1.2355x over previous
import functools
import numpy as np
import jax
import jax.numpy as jnp
from jax import lax
from jax.experimental import pallas as pl
from jax.experimental.pallas import tpu as pltpu

_HEADS = 8
_KNN = 16
_ROWS = 256
_SUB = 8


def _proj_kernel(x_ref, pd_ref, nrm_ref, cur_ref, den_ref, lin_ref,
                 wq_ref, wk_ref, wv_ref, gw_ref, gb_ref,
                 q_ref, k_ref, v_ref, rowop_ref, colop_ref, gbias_ref):
    xb = x_ref[0]
    q_ref[0] = jnp.dot(xb, wq_ref[...], preferred_element_type=jnp.float32)
    k_ref[0] = jnp.dot(xb, wk_ref[...], preferred_element_type=jnp.float32)
    v_ref[0] = jnp.dot(xb, wv_ref[...], preferred_element_type=jnp.float32)
    geo = jnp.concatenate([pd_ref[0], nrm_ref[0], cur_ref[0],
                           den_ref[0], lin_ref[0]], axis=1)
    r = geo.shape[0]
    sq = jnp.sum(geo * geo, axis=1, keepdims=True)
    zeros7 = jnp.zeros((r, 7), jnp.float32)
    zeros6 = jnp.zeros((r, 6), jnp.float32)
    rowop_ref[0] = jnp.concatenate([-2.0 * geo, zeros7], axis=1)
    colop_ref[0] = jnp.concatenate([geo, sq, zeros6], axis=1)
    geo16 = jnp.concatenate([geo, zeros7], axis=1)
    gbias_ref[0] = (jnp.dot(geo16, gw_ref[...],
                            preferred_element_type=jnp.float32)
                    + gb_ref[...]) * jnp.float32(
                        np.sqrt(x_ref.shape[2] / _HEADS))


def _attn_kernel(q_ref, k_ref, v_ref, rowop_ref, colop_ref, gbias_ref,
                 wo_ref, o_ref, dscr, mscr):
    hi = jax.lax.Precision.HIGHEST
    rb = q_ref.shape[1]
    nn = k_ref.shape[1]
    h = _HEADS
    dh = q_ref.shape[2] // h
    g = lax.dot_general(rowop_ref[0], colop_ref[0], (((1,), (1,)), ((), ())),
                        preferred_element_type=jnp.float32)
    e9 = (lax.broadcasted_iota(jnp.int32, (1, 16), 1) == 9
          ).astype(jnp.float32)
    sqrow = lax.dot_general(e9, colop_ref[0], (((1,), (1,)), ((), ())),
                            precision=hi, preferred_element_type=jnp.float32)
    dscr[...] = g + sqrow

    inf = jnp.float32(np.inf)

    def sub(j, carry):
        dw = dscr[pl.ds(j * _SUB, _SUB), :]
        m = jnp.min(dw, axis=1, keepdims=True)
        for _ in range(_KNN - 1):
            dw = jnp.where(dw <= m, inf, dw)
            m = jnp.min(dw, axis=1, keepdims=True)
        orig = dscr[pl.ds(j * _SUB, _SUB), :]
        mscr[pl.ds(j * _SUB, _SUB), :] = jnp.where(orig <= m, 0.0, -1e30)
        return carry

    lax.fori_loop(0, rb // _SUB, sub, 0, unroll=8)

    q = q_ref[0]
    k = k_ref[0]
    v = v_ref[0]
    gbias = gbias_ref[0]
    scale = jnp.float32(1.0 / np.sqrt(dh))
    ones_r = jnp.ones((rb, 1), jnp.float32)
    ones_n = jnp.ones((nn, 1), jnp.float32)
    msk = mscr[...]
    outs = []
    for hh in range(h):
        qa = jnp.concatenate([q[:, hh * dh:(hh + 1) * dh], ones_r], axis=1)
        ka = jnp.concatenate([k[:, hh * dh:(hh + 1) * dh],
                              gbias[:, hh:hh + 1]], axis=1)
        va = jnp.concatenate([v[:, hh * dh:(hh + 1) * dh], ones_n], axis=1)
        t = lax.dot_general(qa, ka, (((1,), (1,)), ((), ())),
                            preferred_element_type=jnp.float32)
        p = jnp.exp((t + msk) * scale)
        oa = lax.dot_general(p, va, (((1,), (0,)), ((), ())),
                             preferred_element_type=jnp.float32)
        outs.append(oa[:, :dh] / oa[:, dh:dh + 1])
    ob = jnp.concatenate(outs, axis=1)
    o_ref[0] = jnp.dot(ob, wo_ref[...], preferred_element_type=jnp.float32)


@jax.jit
def kernel(x, principal_dir, curvature, density, normals, linearity,
           Wq, Wk, Wv, Wo, geo_w, geo_b):
    b, n, d = x.shape
    r = _ROWS
    nb = n // r
    f32 = jnp.float32
    gw16 = jnp.zeros((16, _HEADS), f32).at[:geo_w.shape[0]].set(geo_w)
    gb2 = geo_b.reshape(1, _HEADS)

    grid = (b, nb)
    row3 = lambda bi, i: (bi, i, 0)
    full3 = lambda bi, i: (bi, 0, 0)
    wmap = lambda bi, i: (0, 0)

    q, k, v, rowop, colop, gbias = pl.pallas_call(
        _proj_kernel,
        grid=grid,
        in_specs=[
            pl.BlockSpec((1, r, d), row3),
            pl.BlockSpec((1, r, 3), row3),
            pl.BlockSpec((1, r, 3), row3),
            pl.BlockSpec((1, r, 1), row3),
            pl.BlockSpec((1, r, 1), row3),
            pl.BlockSpec((1, r, 1), row3),
            pl.BlockSpec((d, d), wmap),
            pl.BlockSpec((d, d), wmap),
            pl.BlockSpec((d, d), wmap),
            pl.BlockSpec((16, _HEADS), wmap),
            pl.BlockSpec((1, _HEADS), wmap),
        ],
        out_specs=[
            pl.BlockSpec((1, r, d), row3),
            pl.BlockSpec((1, r, d), row3),
            pl.BlockSpec((1, r, d), row3),
            pl.BlockSpec((1, r, 16), row3),
            pl.BlockSpec((1, r, 16), row3),
            pl.BlockSpec((1, r, _HEADS), row3),
        ],
        out_shape=[
            jax.ShapeDtypeStruct((b, n, d), f32),
            jax.ShapeDtypeStruct((b, n, d), f32),
            jax.ShapeDtypeStruct((b, n, d), f32),
            jax.ShapeDtypeStruct((b, n, 16), f32),
            jax.ShapeDtypeStruct((b, n, 16), f32),
            jax.ShapeDtypeStruct((b, n, _HEADS), f32),
        ],
        compiler_params=pltpu.CompilerParams(
            dimension_semantics=("parallel", "parallel")),
    )(x, principal_dir, normals, curvature, density, linearity,
      Wq, Wk, Wv, gw16, gb2)

    out = pl.pallas_call(
        _attn_kernel,
        grid=grid,
        in_specs=[
            pl.BlockSpec((1, r, d), row3),
            pl.BlockSpec((1, n, d), full3),
            pl.BlockSpec((1, n, d), full3),
            pl.BlockSpec((1, r, 16), row3),
            pl.BlockSpec((1, n, 16), full3),
            pl.BlockSpec((1, n, _HEADS), full3),
            pl.BlockSpec((d, d), wmap),
        ],
        out_specs=pl.BlockSpec((1, r, d), row3),
        out_shape=jax.ShapeDtypeStruct((b, n, d), f32),
        scratch_shapes=[
            pltpu.VMEM((r, n), f32),
            pltpu.VMEM((r, n), f32),
        ],
        compiler_params=pltpu.CompilerParams(
            dimension_semantics=("parallel", "arbitrary")),
    )(q, k, v, rowop, colop, gbias, Wo)
    return out


# speedup vs baseline: 22.2650x; 1.0096x over previous
import functools
import numpy as np
import jax
import jax.numpy as jnp
from jax import lax
from jax.experimental import pallas as pl
from jax.experimental.pallas import tpu as pltpu

_HEADS = 8
_KNN = 16
_ROWS = 256
_SUB = 8


def _proj_kernel(x_ref, pd_ref, nrm_ref, cur_ref, den_ref, lin_ref,
                 wq_ref, wk_ref, wv_ref, gw_ref, gb_ref,
                 q_ref, k_ref, v_ref, rowop_ref, colop_ref, gbias_ref):
    xb = x_ref[0]
    q_ref[0] = jnp.dot(xb, wq_ref[...], preferred_element_type=jnp.float32)
    k_ref[0] = jnp.dot(xb, wk_ref[...], preferred_element_type=jnp.float32)
    v_ref[0] = jnp.dot(xb, wv_ref[...], preferred_element_type=jnp.float32)
    geo = jnp.concatenate([pd_ref[0], nrm_ref[0], cur_ref[0],
                           den_ref[0], lin_ref[0]], axis=1)
    r = geo.shape[0]
    sq = jnp.sum(geo * geo, axis=1, keepdims=True)
    zeros7 = jnp.zeros((r, 7), jnp.float32)
    zeros6 = jnp.zeros((r, 6), jnp.float32)
    rowop_ref[0] = jnp.concatenate([-2.0 * geo, zeros7], axis=1)
    colop_ref[0] = jnp.concatenate([geo, sq, zeros6], axis=1)
    geo16 = jnp.concatenate([geo, zeros7], axis=1)
    gbias_ref[0] = (jnp.dot(geo16, gw_ref[...],
                            preferred_element_type=jnp.float32)
                    + gb_ref[...]) * jnp.float32(
                        np.sqrt(x_ref.shape[2] / _HEADS))


def _attn_kernel(q_ref, k_ref, v_ref, rowop_ref, colop_ref, gbias_ref,
                 wo_ref, o_ref, dscr, mscr):
    hi = jax.lax.Precision.HIGHEST
    rb = q_ref.shape[1]
    nn = k_ref.shape[1]
    h = _HEADS
    dh = q_ref.shape[2] // h
    g = lax.dot_general(rowop_ref[0], colop_ref[0], (((1,), (1,)), ((), ())),
                        preferred_element_type=jnp.float32)
    e9 = (lax.broadcasted_iota(jnp.int32, (1, 16), 1) == 9
          ).astype(jnp.float32)
    sqrow = lax.dot_general(e9, colop_ref[0], (((1,), (1,)), ((), ())),
                            precision=hi, preferred_element_type=jnp.float32)
    dscr[...] = g + sqrow

    inf = jnp.float32(np.inf)

    def sub(j, carry):
        dw = dscr[pl.ds(j * _SUB, _SUB), :]
        m = jnp.min(dw, axis=1, keepdims=True)
        for _ in range(_KNN - 1):
            dw = jnp.where(dw <= m, inf, dw)
            m = jnp.min(dw, axis=1, keepdims=True)
        orig = dscr[pl.ds(j * _SUB, _SUB), :]
        mscr[pl.ds(j * _SUB, _SUB), :] = jnp.where(orig <= m, 0.0, -1e30)
        return carry

    lax.fori_loop(0, rb // _SUB, sub, 0, unroll=16)

    q = q_ref[0]
    k = k_ref[0]
    v = v_ref[0]
    gbias = gbias_ref[0]
    scale = jnp.float32(1.0 / np.sqrt(dh))
    ones_r = jnp.ones((rb, 1), jnp.float32)
    ones_n = jnp.ones((nn, 1), jnp.float32)
    msk = mscr[...]
    outs = []
    for hh in range(h):
        qa = jnp.concatenate([q[:, hh * dh:(hh + 1) * dh], ones_r], axis=1)
        ka = jnp.concatenate([k[:, hh * dh:(hh + 1) * dh],
                              gbias[:, hh:hh + 1]], axis=1)
        va = jnp.concatenate([v[:, hh * dh:(hh + 1) * dh], ones_n], axis=1)
        t = lax.dot_general(qa, ka, (((1,), (1,)), ((), ())),
                            preferred_element_type=jnp.float32)
        p = jnp.exp((t + msk) * scale)
        oa = lax.dot_general(p, va, (((1,), (0,)), ((), ())),
                             preferred_element_type=jnp.float32)
        outs.append(oa[:, :dh] / oa[:, dh:dh + 1])
    ob = jnp.concatenate(outs, axis=1)
    o_ref[0] = jnp.dot(ob, wo_ref[...], preferred_element_type=jnp.float32)


@jax.jit
def kernel(x, principal_dir, curvature, density, normals, linearity,
           Wq, Wk, Wv, Wo, geo_w, geo_b):
    b, n, d = x.shape
    r = _ROWS
    nb = n // r
    f32 = jnp.float32
    gw16 = jnp.zeros((16, _HEADS), f32).at[:geo_w.shape[0]].set(geo_w)
    gb2 = geo_b.reshape(1, _HEADS)

    grid = (b, nb)
    row3 = lambda bi, i: (bi, i, 0)
    full3 = lambda bi, i: (bi, 0, 0)
    wmap = lambda bi, i: (0, 0)

    q, k, v, rowop, colop, gbias = pl.pallas_call(
        _proj_kernel,
        grid=grid,
        in_specs=[
            pl.BlockSpec((1, r, d), row3),
            pl.BlockSpec((1, r, 3), row3),
            pl.BlockSpec((1, r, 3), row3),
            pl.BlockSpec((1, r, 1), row3),
            pl.BlockSpec((1, r, 1), row3),
            pl.BlockSpec((1, r, 1), row3),
            pl.BlockSpec((d, d), wmap),
            pl.BlockSpec((d, d), wmap),
            pl.BlockSpec((d, d), wmap),
            pl.BlockSpec((16, _HEADS), wmap),
            pl.BlockSpec((1, _HEADS), wmap),
        ],
        out_specs=[
            pl.BlockSpec((1, r, d), row3),
            pl.BlockSpec((1, r, d), row3),
            pl.BlockSpec((1, r, d), row3),
            pl.BlockSpec((1, r, 16), row3),
            pl.BlockSpec((1, r, 16), row3),
            pl.BlockSpec((1, r, _HEADS), row3),
        ],
        out_shape=[
            jax.ShapeDtypeStruct((b, n, d), f32),
            jax.ShapeDtypeStruct((b, n, d), f32),
            jax.ShapeDtypeStruct((b, n, d), f32),
            jax.ShapeDtypeStruct((b, n, 16), f32),
            jax.ShapeDtypeStruct((b, n, 16), f32),
            jax.ShapeDtypeStruct((b, n, _HEADS), f32),
        ],
        compiler_params=pltpu.CompilerParams(
            dimension_semantics=("parallel", "parallel")),
    )(x, principal_dir, normals, curvature, density, linearity,
      Wq, Wk, Wv, gw16, gb2)

    out = pl.pallas_call(
        _attn_kernel,
        grid=grid,
        in_specs=[
            pl.BlockSpec((1, r, d), row3),
            pl.BlockSpec((1, n, d), full3),
            pl.BlockSpec((1, n, d), full3),
            pl.BlockSpec((1, r, 16), row3),
            pl.BlockSpec((1, n, 16), full3),
            pl.BlockSpec((1, n, _HEADS), full3),
            pl.BlockSpec((d, d), wmap),
        ],
        out_specs=pl.BlockSpec((1, r, d), row3),
        out_shape=jax.ShapeDtypeStruct((b, n, d), f32),
        scratch_shapes=[
            pltpu.VMEM((r, n), f32),
            pltpu.VMEM((r, n), f32),
        ],
        compiler_params=pltpu.CompilerParams(
            dimension_semantics=("parallel", "arbitrary")),
    )(q, k, v, rowop, colop, gbias, Wo)
    return out


# speedup vs baseline: 22.2743x; 1.0004x over previous
import functools
import numpy as np
import jax
import jax.numpy as jnp
from jax import lax
from jax.experimental import pallas as pl
from jax.experimental.pallas import tpu as pltpu

_HEADS = 8
_KNN = 16
_ROWS = 256
_SUB = 8


def _proj_kernel(x_ref, pd_ref, nrm_ref, cur_ref, den_ref, lin_ref,
                 wq_ref, wk_ref, wv_ref, gw_ref, gb_ref,
                 q_ref, k_ref, v_ref, rowop_ref, colop_ref, gbias_ref):
    xb = x_ref[0]
    q_ref[0] = jnp.dot(xb, wq_ref[...],
                       preferred_element_type=jnp.float32
                       ).astype(jnp.bfloat16)
    k_ref[0] = jnp.dot(xb, wk_ref[...],
                       preferred_element_type=jnp.float32
                       ).astype(jnp.bfloat16)
    v_ref[0] = jnp.dot(xb, wv_ref[...],
                       preferred_element_type=jnp.float32
                       ).astype(jnp.bfloat16)
    geo = jnp.concatenate([pd_ref[0], nrm_ref[0], cur_ref[0],
                           den_ref[0], lin_ref[0]], axis=1)
    r = geo.shape[0]
    sq = jnp.sum(geo * geo, axis=1, keepdims=True)
    zeros7 = jnp.zeros((r, 7), jnp.float32)
    zeros6 = jnp.zeros((r, 6), jnp.float32)
    rowop_ref[0] = jnp.concatenate([-2.0 * geo, zeros7], axis=1)
    colop_ref[0] = jnp.concatenate([geo, sq, zeros6], axis=1)
    geo16 = jnp.concatenate([geo, zeros7], axis=1)
    gbias_ref[0] = ((jnp.dot(geo16, gw_ref[...],
                             preferred_element_type=jnp.float32)
                     + gb_ref[...]) * jnp.float32(
                        np.sqrt(x_ref.shape[2] / _HEADS))
                    ).astype(jnp.bfloat16)


def _attn_kernel(q_ref, k_ref, v_ref, rowop_ref, colop_ref, gbias_ref,
                 wo_ref, o_ref, dscr, mscr):
    hi = jax.lax.Precision.HIGHEST
    rb = q_ref.shape[1]
    nn = k_ref.shape[1]
    h = _HEADS
    dh = q_ref.shape[2] // h
    g = lax.dot_general(rowop_ref[0], colop_ref[0], (((1,), (1,)), ((), ())),
                        preferred_element_type=jnp.float32)
    e9 = (lax.broadcasted_iota(jnp.int32, (1, 16), 1) == 9
          ).astype(jnp.float32)
    sqrow = lax.dot_general(e9, colop_ref[0], (((1,), (1,)), ((), ())),
                            precision=hi, preferred_element_type=jnp.float32)
    dscr[...] = g + sqrow

    inf = jnp.float32(np.inf)

    def sub(j, carry):
        dw = dscr[pl.ds(j * _SUB, _SUB), :]
        m = jnp.min(dw, axis=1, keepdims=True)
        for _ in range(_KNN - 1):
            dw = jnp.where(dw <= m, inf, dw)
            m = jnp.min(dw, axis=1, keepdims=True)
        orig = dscr[pl.ds(j * _SUB, _SUB), :]
        mscr[pl.ds(j * _SUB, _SUB), :] = jnp.where(orig <= m, 0.0, -1e30)
        return carry

    lax.fori_loop(0, rb // _SUB, sub, 0, unroll=16)

    q = q_ref[0]
    k = k_ref[0]
    v = v_ref[0]
    gbias = gbias_ref[0]
    scale = jnp.float32(1.0 / np.sqrt(dh))
    ones_r = jnp.ones((rb, 1), jnp.bfloat16)
    ones_n = jnp.ones((nn, 1), jnp.bfloat16)
    msk = mscr[...]
    outs = []
    for hh in range(h):
        qa = jnp.concatenate([q[:, hh * dh:(hh + 1) * dh], ones_r], axis=1)
        ka = jnp.concatenate([k[:, hh * dh:(hh + 1) * dh],
                              gbias[:, hh:hh + 1]], axis=1)
        va = jnp.concatenate([v[:, hh * dh:(hh + 1) * dh], ones_n], axis=1)
        t = lax.dot_general(qa, ka, (((1,), (1,)), ((), ())),
                            preferred_element_type=jnp.float32)
        p = jnp.exp((t + msk) * scale).astype(jnp.bfloat16)
        oa = lax.dot_general(p, va, (((1,), (0,)), ((), ())),
                             preferred_element_type=jnp.float32)
        outs.append(oa[:, :dh] / oa[:, dh:dh + 1])
    ob = jnp.concatenate(outs, axis=1)
    o_ref[0] = jnp.dot(ob, wo_ref[...], preferred_element_type=jnp.float32)


@jax.jit
def kernel(x, principal_dir, curvature, density, normals, linearity,
           Wq, Wk, Wv, Wo, geo_w, geo_b):
    b, n, d = x.shape
    r = _ROWS
    nb = n // r
    f32 = jnp.float32
    gw16 = jnp.zeros((16, _HEADS), f32).at[:geo_w.shape[0]].set(geo_w)
    gb2 = geo_b.reshape(1, _HEADS)

    grid = (b, nb)
    row3 = lambda bi, i: (bi, i, 0)
    full3 = lambda bi, i: (bi, 0, 0)
    wmap = lambda bi, i: (0, 0)

    q, k, v, rowop, colop, gbias = pl.pallas_call(
        _proj_kernel,
        grid=grid,
        in_specs=[
            pl.BlockSpec((1, r, d), row3),
            pl.BlockSpec((1, r, 3), row3),
            pl.BlockSpec((1, r, 3), row3),
            pl.BlockSpec((1, r, 1), row3),
            pl.BlockSpec((1, r, 1), row3),
            pl.BlockSpec((1, r, 1), row3),
            pl.BlockSpec((d, d), wmap),
            pl.BlockSpec((d, d), wmap),
            pl.BlockSpec((d, d), wmap),
            pl.BlockSpec((16, _HEADS), wmap),
            pl.BlockSpec((1, _HEADS), wmap),
        ],
        out_specs=[
            pl.BlockSpec((1, r, d), row3),
            pl.BlockSpec((1, r, d), row3),
            pl.BlockSpec((1, r, d), row3),
            pl.BlockSpec((1, r, 16), row3),
            pl.BlockSpec((1, r, 16), row3),
            pl.BlockSpec((1, r, _HEADS), row3),
        ],
        out_shape=[
            jax.ShapeDtypeStruct((b, n, d), jnp.bfloat16),
            jax.ShapeDtypeStruct((b, n, d), jnp.bfloat16),
            jax.ShapeDtypeStruct((b, n, d), jnp.bfloat16),
            jax.ShapeDtypeStruct((b, n, 16), f32),
            jax.ShapeDtypeStruct((b, n, 16), f32),
            jax.ShapeDtypeStruct((b, n, _HEADS), jnp.bfloat16),
        ],
        compiler_params=pltpu.CompilerParams(
            dimension_semantics=("parallel", "parallel")),
    )(x, principal_dir, normals, curvature, density, linearity,
      Wq, Wk, Wv, gw16, gb2)

    out = pl.pallas_call(
        _attn_kernel,
        grid=grid,
        in_specs=[
            pl.BlockSpec((1, r, d), row3),
            pl.BlockSpec((1, n, d), full3),
            pl.BlockSpec((1, n, d), full3),
            pl.BlockSpec((1, r, 16), row3),
            pl.BlockSpec((1, n, 16), full3),
            pl.BlockSpec((1, n, _HEADS), full3),
            pl.BlockSpec((d, d), wmap),
        ],
        out_specs=pl.BlockSpec((1, r, d), row3),
        out_shape=jax.ShapeDtypeStruct((b, n, d), f32),
        scratch_shapes=[
            pltpu.VMEM((r, n), f32),
            pltpu.VMEM((r, n), f32),
        ],
        compiler_params=pltpu.CompilerParams(
            dimension_semantics=("parallel", "arbitrary")),
    )(q, k, v, rowop, colop, gbias, Wo)
    return out


# speedup vs baseline: 23.1227x; 1.0381x over previous
import functools
import numpy as np
import jax
import jax.numpy as jnp
from jax import lax
from jax.experimental import pallas as pl
from jax.experimental.pallas import tpu as pltpu

_HEADS = 8
_KNN = 16
_ROWS = 256
_SUB = 8


def _proj_kernel(x_ref, pd_ref, nrm_ref, cur_ref, den_ref, lin_ref,
                 wq_ref, wk_ref, wv_ref, gw_ref, gb_ref,
                 q_ref, k_ref, v_ref, rowop_ref, colop_ref, gbias_ref):
    xb = x_ref[0]
    q_ref[0] = jnp.dot(xb, wq_ref[...],
                       preferred_element_type=jnp.float32
                       ).astype(jnp.bfloat16)
    k_ref[0] = jnp.dot(xb, wk_ref[...],
                       preferred_element_type=jnp.float32
                       ).astype(jnp.bfloat16)
    v_ref[0] = jnp.dot(xb, wv_ref[...],
                       preferred_element_type=jnp.float32
                       ).astype(jnp.bfloat16)
    geo = jnp.concatenate([pd_ref[0], nrm_ref[0], cur_ref[0],
                           den_ref[0], lin_ref[0]], axis=1)
    r = geo.shape[0]
    sq = jnp.sum(geo * geo, axis=1, keepdims=True)
    zeros7 = jnp.zeros((r, 7), jnp.float32)
    zeros6 = jnp.zeros((r, 6), jnp.float32)
    rowop_ref[0] = jnp.concatenate([-2.0 * geo, zeros7], axis=1)
    colop_ref[0] = jnp.concatenate([geo, sq, zeros6], axis=1)
    geo16 = jnp.concatenate([geo, zeros7], axis=1)
    gbias_ref[0] = ((jnp.dot(geo16, gw_ref[...],
                             preferred_element_type=jnp.float32)
                     + gb_ref[...]) * jnp.float32(
                        np.sqrt(x_ref.shape[2] / _HEADS))
                    ).astype(jnp.bfloat16)


def _attn_kernel(q_ref, k_ref, v_ref, rowop_ref, colop_ref, gbias_ref,
                 wo_ref, o_ref, dscr, mscr):
    hi = jax.lax.Precision.HIGHEST
    rb = q_ref.shape[1]
    nn = k_ref.shape[1]
    h = _HEADS
    dh = q_ref.shape[2] // h
    g = lax.dot_general(rowop_ref[0], colop_ref[0], (((1,), (1,)), ((), ())),
                        preferred_element_type=jnp.float32)
    e9 = (lax.broadcasted_iota(jnp.int32, (1, 16), 1) == 9
          ).astype(jnp.float32)
    sqrow = lax.dot_general(e9, colop_ref[0], (((1,), (1,)), ((), ())),
                            precision=hi, preferred_element_type=jnp.float32)
    dscr[...] = g + sqrow

    inf = jnp.float32(np.inf)

    def sub(j, carry):
        dw = dscr[pl.ds(j * _SUB, _SUB), :]
        m = jnp.min(dw, axis=1, keepdims=True)
        for _ in range(_KNN - 1):
            dw = jnp.where(dw <= m, inf, dw)
            m = jnp.min(dw, axis=1, keepdims=True)
        orig = dscr[pl.ds(j * _SUB, _SUB), :]
        mscr[pl.ds(j * _SUB, _SUB), :] = jnp.where(orig <= m, 0.0, -1e30)
        return carry

    lax.fori_loop(0, rb // _SUB, sub, 0, unroll=16)

    q = q_ref[0]
    k = k_ref[0]
    v = v_ref[0]
    gbias = gbias_ref[0]
    scale = jnp.float32(1.0 / np.sqrt(dh))
    ones_r = jnp.ones((rb, 1), jnp.bfloat16)
    ones_n = jnp.ones((nn, 1), jnp.bfloat16)
    msk = mscr[...]
    outs = []
    for hh in range(h):
        qa = jnp.concatenate([q[:, hh * dh:(hh + 1) * dh], ones_r], axis=1)
        ka = jnp.concatenate([k[:, hh * dh:(hh + 1) * dh],
                              gbias[:, hh:hh + 1]], axis=1)
        va = jnp.concatenate([v[:, hh * dh:(hh + 1) * dh], ones_n], axis=1)
        t = lax.dot_general(qa, ka, (((1,), (1,)), ((), ())),
                            preferred_element_type=jnp.float32)
        c2 = jnp.float32(scale * np.log2(np.e))
        p = jnp.exp2((t + msk) * c2).astype(jnp.bfloat16)
        oa = lax.dot_general(p, va, (((1,), (0,)), ((), ())),
                             preferred_element_type=jnp.float32)
        outs.append(oa[:, :dh] / oa[:, dh:dh + 1])
    ob = jnp.concatenate(outs, axis=1)
    o_ref[0] = jnp.dot(ob, wo_ref[...], preferred_element_type=jnp.float32)


@jax.jit
def kernel(x, principal_dir, curvature, density, normals, linearity,
           Wq, Wk, Wv, Wo, geo_w, geo_b):
    b, n, d = x.shape
    r = _ROWS
    nb = n // r
    f32 = jnp.float32
    gw16 = jnp.zeros((16, _HEADS), f32).at[:geo_w.shape[0]].set(geo_w)
    gb2 = geo_b.reshape(1, _HEADS)

    grid = (b, nb)
    row3 = lambda bi, i: (bi, i, 0)
    full3 = lambda bi, i: (bi, 0, 0)
    wmap = lambda bi, i: (0, 0)

    q, k, v, rowop, colop, gbias = pl.pallas_call(
        _proj_kernel,
        grid=grid,
        in_specs=[
            pl.BlockSpec((1, r, d), row3),
            pl.BlockSpec((1, r, 3), row3),
            pl.BlockSpec((1, r, 3), row3),
            pl.BlockSpec((1, r, 1), row3),
            pl.BlockSpec((1, r, 1), row3),
            pl.BlockSpec((1, r, 1), row3),
            pl.BlockSpec((d, d), wmap),
            pl.BlockSpec((d, d), wmap),
            pl.BlockSpec((d, d), wmap),
            pl.BlockSpec((16, _HEADS), wmap),
            pl.BlockSpec((1, _HEADS), wmap),
        ],
        out_specs=[
            pl.BlockSpec((1, r, d), row3),
            pl.BlockSpec((1, r, d), row3),
            pl.BlockSpec((1, r, d), row3),
            pl.BlockSpec((1, r, 16), row3),
            pl.BlockSpec((1, r, 16), row3),
            pl.BlockSpec((1, r, _HEADS), row3),
        ],
        out_shape=[
            jax.ShapeDtypeStruct((b, n, d), jnp.bfloat16),
            jax.ShapeDtypeStruct((b, n, d), jnp.bfloat16),
            jax.ShapeDtypeStruct((b, n, d), jnp.bfloat16),
            jax.ShapeDtypeStruct((b, n, 16), f32),
            jax.ShapeDtypeStruct((b, n, 16), f32),
            jax.ShapeDtypeStruct((b, n, _HEADS), jnp.bfloat16),
        ],
        compiler_params=pltpu.CompilerParams(
            dimension_semantics=("parallel", "parallel")),
    )(x, principal_dir, normals, curvature, density, linearity,
      Wq, Wk, Wv, gw16, gb2)

    out = pl.pallas_call(
        _attn_kernel,
        grid=grid,
        in_specs=[
            pl.BlockSpec((1, r, d), row3),
            pl.BlockSpec((1, n, d), full3),
            pl.BlockSpec((1, n, d), full3),
            pl.BlockSpec((1, r, 16), row3),
            pl.BlockSpec((1, n, 16), full3),
            pl.BlockSpec((1, n, _HEADS), full3),
            pl.BlockSpec((d, d), wmap),
        ],
        out_specs=pl.BlockSpec((1, r, d), row3),
        out_shape=jax.ShapeDtypeStruct((b, n, d), f32),
        scratch_shapes=[
            pltpu.VMEM((r, n), f32),
            pltpu.VMEM((r, n), f32),
        ],
        compiler_params=pltpu.CompilerParams(
            dimension_semantics=("parallel", "arbitrary")),
    )(q, k, v, rowop, colop, gbias, Wo)
    return out


# speedup vs baseline: 24.8034x; 1.0727x over previous
import functools
import numpy as np
import jax
import jax.numpy as jnp
from jax import lax
from jax.experimental import pallas as pl
from jax.experimental.pallas import tpu as pltpu

_HEADS = 8
_KNN = 16
_ROWS = 512
_SUB = 8


def _proj_kernel(x_ref, pd_ref, nrm_ref, cur_ref, den_ref, lin_ref,
                 wq_ref, wk_ref, wv_ref, gw_ref, gb_ref,
                 q_ref, k_ref, v_ref, rowop_ref, colop_ref, gbias_ref):
    xb = x_ref[0]
    q_ref[0] = jnp.dot(xb, wq_ref[...],
                       preferred_element_type=jnp.float32
                       ).astype(jnp.bfloat16)
    k_ref[0] = jnp.dot(xb, wk_ref[...],
                       preferred_element_type=jnp.float32
                       ).astype(jnp.bfloat16)
    v_ref[0] = jnp.dot(xb, wv_ref[...],
                       preferred_element_type=jnp.float32
                       ).astype(jnp.bfloat16)
    geo = jnp.concatenate([pd_ref[0], nrm_ref[0], cur_ref[0],
                           den_ref[0], lin_ref[0]], axis=1)
    r = geo.shape[0]
    sq = jnp.sum(geo * geo, axis=1, keepdims=True)
    zeros7 = jnp.zeros((r, 7), jnp.float32)
    zeros6 = jnp.zeros((r, 6), jnp.float32)
    rowop_ref[0] = jnp.concatenate([-2.0 * geo, zeros7], axis=1)
    colop_ref[0] = jnp.concatenate([geo, sq, zeros6], axis=1)
    geo16 = jnp.concatenate([geo, zeros7], axis=1)
    gbias_ref[0] = ((jnp.dot(geo16, gw_ref[...],
                             preferred_element_type=jnp.float32)
                     + gb_ref[...]) * jnp.float32(
                        np.sqrt(x_ref.shape[2] / _HEADS))
                    ).astype(jnp.bfloat16)


def _attn_kernel(q_ref, k_ref, v_ref, rowop_ref, colop_ref, gbias_ref,
                 wo_ref, o_ref, dscr, mscr):
    hi = jax.lax.Precision.HIGHEST
    rb = q_ref.shape[1]
    nn = k_ref.shape[1]
    h = _HEADS
    dh = q_ref.shape[2] // h
    g = lax.dot_general(rowop_ref[0], colop_ref[0], (((1,), (1,)), ((), ())),
                        preferred_element_type=jnp.float32)
    e9 = (lax.broadcasted_iota(jnp.int32, (1, 16), 1) == 9
          ).astype(jnp.float32)
    sqrow = lax.dot_general(e9, colop_ref[0], (((1,), (1,)), ((), ())),
                            precision=hi, preferred_element_type=jnp.float32)
    dscr[...] = g + sqrow

    inf = jnp.float32(np.inf)

    def sub(j, carry):
        dw = dscr[pl.ds(j * _SUB, _SUB), :]
        m = jnp.min(dw, axis=1, keepdims=True)
        for _ in range(_KNN - 1):
            dw = jnp.where(dw <= m, inf, dw)
            m = jnp.min(dw, axis=1, keepdims=True)
        orig = dscr[pl.ds(j * _SUB, _SUB), :]
        mscr[pl.ds(j * _SUB, _SUB), :] = jnp.where(orig <= m, 0.0, -1e30)
        return carry

    lax.fori_loop(0, rb // _SUB, sub, 0, unroll=16)

    q = q_ref[0]
    k = k_ref[0]
    v = v_ref[0]
    gbias = gbias_ref[0]
    scale = jnp.float32(1.0 / np.sqrt(dh))
    ones_r = jnp.ones((rb, 1), jnp.bfloat16)
    ones_n = jnp.ones((nn, 1), jnp.bfloat16)
    msk = mscr[...]
    outs = []
    for hh in range(h):
        qa = jnp.concatenate([q[:, hh * dh:(hh + 1) * dh], ones_r], axis=1)
        ka = jnp.concatenate([k[:, hh * dh:(hh + 1) * dh],
                              gbias[:, hh:hh + 1]], axis=1)
        va = jnp.concatenate([v[:, hh * dh:(hh + 1) * dh], ones_n], axis=1)
        t = lax.dot_general(qa, ka, (((1,), (1,)), ((), ())),
                            preferred_element_type=jnp.float32)
        c2 = jnp.float32(scale * np.log2(np.e))
        p = jnp.exp2((t + msk) * c2).astype(jnp.bfloat16)
        oa = lax.dot_general(p, va, (((1,), (0,)), ((), ())),
                             preferred_element_type=jnp.float32)
        outs.append(oa[:, :dh] / oa[:, dh:dh + 1])
    ob = jnp.concatenate(outs, axis=1)
    o_ref[0] = jnp.dot(ob, wo_ref[...], preferred_element_type=jnp.float32)


@jax.jit
def kernel(x, principal_dir, curvature, density, normals, linearity,
           Wq, Wk, Wv, Wo, geo_w, geo_b):
    b, n, d = x.shape
    r = _ROWS
    nb = n // r
    f32 = jnp.float32
    gw16 = jnp.zeros((16, _HEADS), f32).at[:geo_w.shape[0]].set(geo_w)
    gb2 = geo_b.reshape(1, _HEADS)

    grid = (b, nb)
    row3 = lambda bi, i: (bi, i, 0)
    full3 = lambda bi, i: (bi, 0, 0)
    wmap = lambda bi, i: (0, 0)

    q, k, v, rowop, colop, gbias = pl.pallas_call(
        _proj_kernel,
        grid=grid,
        in_specs=[
            pl.BlockSpec((1, r, d), row3),
            pl.BlockSpec((1, r, 3), row3),
            pl.BlockSpec((1, r, 3), row3),
            pl.BlockSpec((1, r, 1), row3),
            pl.BlockSpec((1, r, 1), row3),
            pl.BlockSpec((1, r, 1), row3),
            pl.BlockSpec((d, d), wmap),
            pl.BlockSpec((d, d), wmap),
            pl.BlockSpec((d, d), wmap),
            pl.BlockSpec((16, _HEADS), wmap),
            pl.BlockSpec((1, _HEADS), wmap),
        ],
        out_specs=[
            pl.BlockSpec((1, r, d), row3),
            pl.BlockSpec((1, r, d), row3),
            pl.BlockSpec((1, r, d), row3),
            pl.BlockSpec((1, r, 16), row3),
            pl.BlockSpec((1, r, 16), row3),
            pl.BlockSpec((1, r, _HEADS), row3),
        ],
        out_shape=[
            jax.ShapeDtypeStruct((b, n, d), jnp.bfloat16),
            jax.ShapeDtypeStruct((b, n, d), jnp.bfloat16),
            jax.ShapeDtypeStruct((b, n, d), jnp.bfloat16),
            jax.ShapeDtypeStruct((b, n, 16), f32),
            jax.ShapeDtypeStruct((b, n, 16), f32),
            jax.ShapeDtypeStruct((b, n, _HEADS), jnp.bfloat16),
        ],
        compiler_params=pltpu.CompilerParams(
            dimension_semantics=("parallel", "parallel")),
    )(x, principal_dir, normals, curvature, density, linearity,
      Wq, Wk, Wv, gw16, gb2)

    out = pl.pallas_call(
        _attn_kernel,
        grid=grid,
        in_specs=[
            pl.BlockSpec((1, r, d), row3),
            pl.BlockSpec((1, n, d), full3),
            pl.BlockSpec((1, n, d), full3),
            pl.BlockSpec((1, r, 16), row3),
            pl.BlockSpec((1, n, 16), full3),
            pl.BlockSpec((1, n, _HEADS), full3),
            pl.BlockSpec((d, d), wmap),
        ],
        out_specs=pl.BlockSpec((1, r, d), row3),
        out_shape=jax.ShapeDtypeStruct((b, n, d), f32),
        scratch_shapes=[
            pltpu.VMEM((r, n), f32),
            pltpu.VMEM((r, n), f32),
        ],
        compiler_params=pltpu.CompilerParams(
            dimension_semantics=("parallel", "arbitrary")),
    )(q, k, v, rowop, colop, gbias, Wo)
    return out
